# initial kernel scaffold (unmeasured)
import jax
import jax.numpy as jnp
from jax import lax
from jax.experimental import pallas as pl
from jax.experimental.pallas import tpu as pltpu

ROWS = 4096
NCOL = 1024
CH = 256
NMAX = ROWS // CH


def kernel(x, dest):
    p = lax.axis_index("x")
    to_peer = dest != p
    m_send = jnp.sum(to_peer.astype(jnp.int32))
    perm = jnp.argsort(jnp.where(to_peer, 0, 1), stable=True)
    xs = jnp.take(x.astype(jnp.bfloat16), perm, axis=0)

    def body(msend_ref, xs_ref, out_ref, recv_buf, send_sems, recv_sems):
        my_x = lax.axis_index("x")
        my_y = lax.axis_index("y")
        my_z = lax.axis_index("z")
        peer = (1 - my_x, my_y, my_z)
        m_send = msend_ref[0]
        m_self = ROWS - m_send
        incoming = m_send

        barrier = pltpu.get_barrier_semaphore()
        pl.semaphore_signal(
            barrier, inc=1, device_id=peer,
            device_id_type=pl.DeviceIdType.MESH,
        )
        pl.semaphore_wait(barrier, 1)

        rdmas = []
        for k in range(NMAX):
            rdma = pltpu.make_async_remote_copy(
                src_ref=xs_ref.at[pl.ds(k * CH, CH)],
                dst_ref=recv_buf.at[pl.ds(k * CH, CH)],
                send_sem=send_sems.at[k],
                recv_sem=recv_sems.at[k],
                device_id=peer,
                device_id_type=pl.DeviceIdType.MESH,
            )
            rdma.start()
            rdmas.append(rdma)

        off_own = jnp.where(my_x == 0, 0, incoming)
        for k in range(NMAX):
            @pl.when(k * CH < m_self)
            def _():
                sk = jnp.minimum(k * CH, m_self - CH)
                out_ref[pl.ds(off_own + sk, CH), :] = (
                    xs_ref[pl.ds(m_send + sk, CH), :]
                )

        off_in = jnp.where(my_x == 0, ROWS - incoming, 0)
        for k in range(NMAX):
            rdmas[k].wait_recv()

            @pl.when(k * CH < incoming)
            def _():
                sk = jnp.minimum(k * CH, incoming - CH)
                out_ref[pl.ds(off_in + sk, CH), :] = recv_buf[pl.ds(sk, CH), :]

        for k in range(NMAX):
            rdmas[k].wait_send()

    return pl.pallas_call(
        body,
        out_shape=jax.ShapeDtypeStruct((ROWS, NCOL), jnp.bfloat16),
        in_specs=[
            pl.BlockSpec(memory_space=pltpu.SMEM),
            pl.BlockSpec(memory_space=pltpu.VMEM),
        ],
        out_specs=pl.BlockSpec(memory_space=pltpu.VMEM),
        scratch_shapes=[
            pltpu.VMEM((ROWS, NCOL), jnp.bfloat16),
            pltpu.SemaphoreType.DMA((NMAX,)),
            pltpu.SemaphoreType.DMA((NMAX,)),
        ],
        compiler_params=pltpu.CompilerParams(collective_id=0),
    )(m_send.reshape(1), xs)


# baseline (device time: 165592 ns/iter reference)
import jax
import jax.numpy as jnp
from jax import lax
from jax.experimental import pallas as pl
from jax.experimental.pallas import tpu as pltpu

ROWS = 4096
NCOL = 1024
CH = 256
NMAX = ROWS // CH


def _exchange(m_send_arr, xs):

    def body(msend_ref, xs_ref, out_ref, send_sems, recv_sems):
        my_x = lax.axis_index("x")
        my_y = lax.axis_index("y")
        my_z = lax.axis_index("z")
        peer = (1 - my_x, my_y, my_z)
        m_send = msend_ref[0]
        incoming = m_send

        barrier = pltpu.get_barrier_semaphore()
        pl.semaphore_signal(
            barrier, inc=1, device_id=peer,
            device_id_type=pl.DeviceIdType.MESH,
        )
        pl.semaphore_wait(barrier, 1)

        rdmas = []
        for k in range(NMAX):
            rdma = pltpu.make_async_remote_copy(
                src_ref=xs_ref.at[pl.ds(k * CH, CH)],
                dst_ref=out_ref.at[pl.ds(k * CH, CH)],
                send_sem=send_sems.at[k],
                recv_sem=recv_sems.at[k],
                device_id=peer,
                device_id_type=pl.DeviceIdType.MESH,
            )
            rdmas.append(rdma)

            @pl.when(k * CH < m_send)
            def _():
                rdma.start()

        for k in range(NMAX):
            @pl.when(k * CH < incoming)
            def _():
                rdmas[k].wait_recv()

        for k in range(NMAX):
            @pl.when(k * CH < m_send)
            def _():
                rdmas[k].wait_send()

    return pl.pallas_call(
        body,
        out_shape=jax.ShapeDtypeStruct((ROWS, NCOL), jnp.bfloat16),
        in_specs=[
            pl.BlockSpec(memory_space=pltpu.SMEM),
            pl.BlockSpec(memory_space=pltpu.VMEM),
        ],
        out_specs=pl.BlockSpec(memory_space=pltpu.VMEM),
        scratch_shapes=[
            pltpu.SemaphoreType.DMA((NMAX,)),
            pltpu.SemaphoreType.DMA((NMAX,)),
        ],
        compiler_params=pltpu.CompilerParams(collective_id=0),
    )(m_send_arr, xs)


def kernel(x, dest):
    p = lax.axis_index("x")
    to_peer = dest != p
    m_send = jnp.sum(to_peer.astype(jnp.int32))
    perm = jnp.argsort(jnp.where(to_peer, 0, 1), stable=True)
    xs = jnp.take(x.astype(jnp.bfloat16), perm, axis=0)

    recv = _exchange(m_send.reshape(1), xs)

    m_self = ROWS - m_send
    i = jnp.arange(ROWS, dtype=jnp.int32)
    stacked = jnp.concatenate([xs, recv], axis=0)
    idx = jnp.where(
        p == 0,
        jnp.where(i < m_self, m_send + i, ROWS + i - m_self),
        jnp.where(i < m_send, ROWS + i, i),
    )
    return jnp.take(stacked, idx, axis=0)


# device time: 119320 ns/iter; 1.3878x vs baseline; 1.3878x over previous
import jax
import jax.numpy as jnp
from jax import lax
from jax.experimental import pallas as pl
from jax.experimental.pallas import tpu as pltpu

ROWS = 4096
NCOL = 1024
CH = 256
NMAX = ROWS // CH


def _exchange_and_assemble(m_send_arr, xs):

    def body(msend_ref, xs_ref, out_ref, recv_buf, send_sems, recv_sems):
        my_x = lax.axis_index("x")
        my_y = lax.axis_index("y")
        my_z = lax.axis_index("z")
        peer = (1 - my_x, my_y, my_z)
        m_send = msend_ref[0]
        m_self = ROWS - m_send
        incoming = m_send

        barrier = pltpu.get_barrier_semaphore()
        pl.semaphore_signal(
            barrier, inc=1, device_id=peer,
            device_id_type=pl.DeviceIdType.MESH,
        )
        pl.semaphore_wait(barrier, 1)

        rdmas = []
        for k in range(NMAX):
            rdma = pltpu.make_async_remote_copy(
                src_ref=xs_ref.at[pl.ds(k * CH, CH)],
                dst_ref=recv_buf.at[pl.ds(k * CH, CH)],
                send_sem=send_sems.at[k],
                recv_sem=recv_sems.at[k],
                device_id=peer,
                device_id_type=pl.DeviceIdType.MESH,
            )
            rdmas.append(rdma)

            @pl.when(k * CH < m_send)
            def _():
                rdma.start()

        for k in range(NMAX):
            @pl.when(k * CH < incoming)
            def _():
                rdmas[k].wait_recv()

        shift = jnp.where(my_x == 0, m_self, 0)
        xs_r = pltpu.roll(xs_ref[...], shift, 0)
        recv_r = pltpu.roll(recv_buf[...], shift, 0)
        row = lax.broadcasted_iota(jnp.int32, (ROWS, NCOL), 0)
        s = jnp.where(my_x == 0, 1, -1)
        t = jnp.where(my_x == 0, m_self, 1 - incoming)
        own_mask = (row * s) < t
        out_ref[...] = jnp.where(own_mask, xs_r, recv_r)

        for k in range(NMAX):
            @pl.when(k * CH < m_send)
            def _():
                rdmas[k].wait_send()

    return pl.pallas_call(
        body,
        out_shape=jax.ShapeDtypeStruct((ROWS, NCOL), jnp.bfloat16),
        in_specs=[
            pl.BlockSpec(memory_space=pltpu.SMEM),
            pl.BlockSpec(memory_space=pltpu.VMEM),
        ],
        out_specs=pl.BlockSpec(memory_space=pltpu.VMEM),
        scratch_shapes=[
            pltpu.VMEM((ROWS, NCOL), jnp.bfloat16),
            pltpu.SemaphoreType.DMA((NMAX,)),
            pltpu.SemaphoreType.DMA((NMAX,)),
        ],
        compiler_params=pltpu.CompilerParams(
            collective_id=0, vmem_limit_bytes=100 * 1024 * 1024,
        ),
    )(m_send_arr, xs)


def kernel(x, dest):
    p = lax.axis_index("x")
    to_peer = dest != p
    m_send = jnp.sum(to_peer.astype(jnp.int32))
    perm = jnp.argsort(jnp.where(to_peer, 0, 1), stable=True)
    xs = jnp.take(x.astype(jnp.bfloat16), perm, axis=0)
    return _exchange_and_assemble(m_send.reshape(1), xs)


# device time: 102278 ns/iter; 1.6190x vs baseline; 1.1666x over previous
import jax
import jax.numpy as jnp
from jax import lax
from jax.experimental import pallas as pl
from jax.experimental.pallas import tpu as pltpu

ROWS = 4096
NCOL = 1024
CH = 256
NMAX = ROWS // CH


def _a2av(m_arr, x_bf, perm_send, perm_own):
    def body(m_ref, x_ref, psend_ref, pown_ref, out_ref, stage,
             send_sems, recv_sems):
        my_x = lax.axis_index("x")
        my_y = lax.axis_index("y")
        my_z = lax.axis_index("z")
        peer = (1 - my_x, my_y, my_z)
        is0 = my_x == 0
        m = m_ref[0]
        m_self = ROWS - m

        xv = x_ref[...]

        def gen(perm_ref, k):
            ids = perm_ref[pl.ds(k * CH, CH), :]
            oh = (lax.broadcasted_iota(jnp.int32, (CH, ROWS), 1) == ids)
            return lax.dot_general(
                oh.astype(jnp.bfloat16), xv,
                (((1,), (0,)), ((), ())),
                preferred_element_type=jnp.float32,
            ).astype(jnp.bfloat16)

        def send_pred(k):
            return jnp.where(is0, k * CH < m, (k + 1) * CH > m_self)

        def recv_pred(k):
            return jnp.where(is0, (k + 1) * CH > m_self, k * CH < m)

        def own_pure(k):
            return jnp.where(is0, (k + 1) * CH <= m_self, k * CH >= m)

        b = jnp.where(is0, m_self, m)

        def boundary(k):
            return (k * CH < b) & (b < (k + 1) * CH)

        barrier = pltpu.get_barrier_semaphore()
        pl.semaphore_signal(
            barrier, inc=1, device_id=peer,
            device_id_type=pl.DeviceIdType.MESH,
        )
        pl.semaphore_wait(barrier, 1)

        rdmas = []
        for k in range(NMAX):
            rdma = pltpu.make_async_remote_copy(
                src_ref=stage.at[pl.ds(k * CH, CH)],
                dst_ref=out_ref.at[pl.ds(k * CH, CH)],
                send_sem=send_sems.at[k],
                recv_sem=recv_sems.at[k],
                device_id=peer,
                device_id_type=pl.DeviceIdType.MESH,
            )
            rdmas.append(rdma)

            @pl.when(send_pred(k))
            def _():
                stage[pl.ds(k * CH, CH), :] = gen(psend_ref, k)
                rdma.start()

        for k in range(NMAX):
            @pl.when(own_pure(k))
            def _():
                out_ref[pl.ds(k * CH, CH), :] = gen(pown_ref, k)

        for k in range(NMAX):
            @pl.when(recv_pred(k))
            def _():
                rdmas[k].wait_recv()

        row = lax.broadcasted_iota(jnp.int32, (CH, NCOL), 0)
        s = jnp.where(is0, 1, -1)
        t = jnp.where(is0, m_self, 1 - m)
        for k in range(NMAX):
            @pl.when(boundary(k))
            def _():
                rel = (row + k * CH) * s
                out_ref[pl.ds(k * CH, CH), :] = jnp.where(
                    rel < t, gen(pown_ref, k), out_ref[pl.ds(k * CH, CH), :]
                )

        for k in range(NMAX):
            @pl.when(send_pred(k))
            def _():
                rdmas[k].wait_send()

    return pl.pallas_call(
        body,
        out_shape=jax.ShapeDtypeStruct((ROWS, NCOL), jnp.bfloat16),
        in_specs=[
            pl.BlockSpec(memory_space=pltpu.SMEM),
            pl.BlockSpec(memory_space=pltpu.VMEM),
            pl.BlockSpec(memory_space=pltpu.VMEM),
            pl.BlockSpec(memory_space=pltpu.VMEM),
        ],
        out_specs=pl.BlockSpec(memory_space=pltpu.VMEM),
        scratch_shapes=[
            pltpu.VMEM((ROWS, NCOL), jnp.bfloat16),
            pltpu.SemaphoreType.DMA((NMAX,)),
            pltpu.SemaphoreType.DMA((NMAX,)),
        ],
        compiler_params=pltpu.CompilerParams(
            collective_id=0, vmem_limit_bytes=100 * 1024 * 1024,
        ),
    )(m_arr, x_bf, perm_send, perm_own)


def kernel(x, dest):
    p = lax.axis_index("x")
    to_self = (dest == p).astype(jnp.int32)
    m = jnp.sum(1 - to_self)
    order = jnp.argsort(to_self, stable=True).astype(jnp.int32)
    order_r = jnp.roll(order, ROWS - m)
    perm_send = jnp.where(p == 0, order, order_r).reshape(ROWS, 1)
    perm_own = jnp.where(p == 0, order_r, order).reshape(ROWS, 1)
    return _a2av(
        m.reshape(1), x.astype(jnp.bfloat16), perm_send, perm_own
    )


# device time: 96646 ns/iter; 1.7134x vs baseline; 1.0583x over previous
import jax
import jax.numpy as jnp
from jax import lax
from jax.experimental import pallas as pl
from jax.experimental.pallas import tpu as pltpu

ROWS = 4096
NCOL = 1024
CH = 512
NMAX = ROWS // CH


def _a2av(m_arr, x_bf, perm_send, perm_own):
    def body(m_ref, x_ref, psend_ref, pown_ref, out_ref, stage,
             send_sems, recv_sems):
        my_x = lax.axis_index("x")
        my_y = lax.axis_index("y")
        my_z = lax.axis_index("z")
        peer = (1 - my_x, my_y, my_z)
        is0 = my_x == 0
        m = m_ref[0]
        m_self = ROWS - m

        xv = x_ref[...]

        def gen(perm_ref, k):
            ids = perm_ref[pl.ds(k * CH, CH), :]
            oh = (lax.broadcasted_iota(jnp.int32, (CH, ROWS), 1) == ids)
            return lax.dot_general(
                oh.astype(jnp.bfloat16), xv,
                (((1,), (0,)), ((), ())),
                preferred_element_type=jnp.float32,
            ).astype(jnp.bfloat16)

        def send_pred(k):
            return jnp.where(is0, k * CH < m, (k + 1) * CH > m_self)

        def recv_pred(k):
            return jnp.where(is0, (k + 1) * CH > m_self, k * CH < m)

        def own_pure(k):
            return jnp.where(is0, (k + 1) * CH <= m_self, k * CH >= m)

        b = jnp.where(is0, m_self, m)

        def boundary(k):
            return (k * CH < b) & (b < (k + 1) * CH)

        barrier = pltpu.get_barrier_semaphore()
        pl.semaphore_signal(
            barrier, inc=1, device_id=peer,
            device_id_type=pl.DeviceIdType.MESH,
        )
        pl.semaphore_wait(barrier, 1)

        rdmas = []
        for k in range(NMAX):
            rdma = pltpu.make_async_remote_copy(
                src_ref=stage.at[pl.ds(k * CH, CH)],
                dst_ref=out_ref.at[pl.ds(k * CH, CH)],
                send_sem=send_sems.at[k],
                recv_sem=recv_sems.at[k],
                device_id=peer,
                device_id_type=pl.DeviceIdType.MESH,
            )
            rdmas.append(rdma)

            @pl.when(send_pred(k))
            def _():
                stage[pl.ds(k * CH, CH), :] = gen(psend_ref, k)
                rdma.start()

        for k in range(NMAX):
            @pl.when(own_pure(k))
            def _():
                out_ref[pl.ds(k * CH, CH), :] = gen(pown_ref, k)

        for k in range(NMAX):
            @pl.when(recv_pred(k))
            def _():
                rdmas[k].wait_recv()

        row = lax.broadcasted_iota(jnp.int32, (CH, NCOL), 0)
        s = jnp.where(is0, 1, -1)
        t = jnp.where(is0, m_self, 1 - m)
        for k in range(NMAX):
            @pl.when(boundary(k))
            def _():
                rel = (row + k * CH) * s
                out_ref[pl.ds(k * CH, CH), :] = jnp.where(
                    rel < t, gen(pown_ref, k), out_ref[pl.ds(k * CH, CH), :]
                )

        for k in range(NMAX):
            @pl.when(send_pred(k))
            def _():
                rdmas[k].wait_send()

    return pl.pallas_call(
        body,
        out_shape=jax.ShapeDtypeStruct((ROWS, NCOL), jnp.bfloat16),
        in_specs=[
            pl.BlockSpec(memory_space=pltpu.SMEM),
            pl.BlockSpec(memory_space=pltpu.VMEM),
            pl.BlockSpec(memory_space=pltpu.VMEM),
            pl.BlockSpec(memory_space=pltpu.VMEM),
        ],
        out_specs=pl.BlockSpec(memory_space=pltpu.VMEM),
        scratch_shapes=[
            pltpu.VMEM((ROWS, NCOL), jnp.bfloat16),
            pltpu.SemaphoreType.DMA((NMAX,)),
            pltpu.SemaphoreType.DMA((NMAX,)),
        ],
        compiler_params=pltpu.CompilerParams(
            collective_id=0, vmem_limit_bytes=100 * 1024 * 1024,
        ),
    )(m_arr, x_bf, perm_send, perm_own)


def kernel(x, dest):
    p = lax.axis_index("x")
    to_self = (dest == p).astype(jnp.int32)
    m = jnp.sum(1 - to_self)
    order = jnp.argsort(to_self, stable=True).astype(jnp.int32)
    order_r = jnp.roll(order, ROWS - m)
    perm_send = jnp.where(p == 0, order, order_r).reshape(ROWS, 1)
    perm_own = jnp.where(p == 0, order_r, order).reshape(ROWS, 1)
    return _a2av(
        m.reshape(1), x.astype(jnp.bfloat16), perm_send, perm_own
    )
